# baseline (speedup 1.0000x reference)
import functools

import jax
import jax.numpy as jnp
from jax import lax
from jax.experimental import pallas as pl
from jax.experimental.pallas import tpu as pltpu
from jax.experimental.pallas import tpu_sc as plsc

_D = 64
_K = 64
_HW = 1024
_LANES = 16


def _dist_body(z_ref, w_ref, idx_ref, *, K):
    z = z_ref[0]
    w = w_ref[...]
    wsq_row = jnp.sum(w * w, axis=0, keepdims=True)
    wsq = jnp.transpose(wsq_row, (1, 0))
    scores = jax.lax.dot_general(
        w, z, (((0,), (0,)), ((), ())),
        preferred_element_type=jnp.float32)
    zsq = jnp.sum(z * z, axis=0, keepdims=True)
    dist = (zsq - 2.0 * scores) + wsq
    mind = jnp.min(dist, axis=0, keepdims=True)
    iota = jax.lax.broadcasted_iota(jnp.int32, dist.shape, 0)
    cand = jnp.where(dist == mind, iota, K)
    idx = jnp.min(cand, axis=0, keepdims=True)
    idx_ref[0, 0] = idx[0]


def _nearest_indices(z3, weight):
    B = z3.shape[0]
    idx3 = pl.pallas_call(
        functools.partial(_dist_body, K=_K),
        grid=(B,),
        in_specs=[
            pl.BlockSpec((1, _D, _HW), lambda i: (i, 0, 0)),
            pl.BlockSpec((_D, _K), lambda i: (0, 0)),
        ],
        out_specs=pl.BlockSpec((1, 1, _HW), lambda i: (i, 0, 0)),
        out_shape=jax.ShapeDtypeStruct((B, 1, _HW), jnp.int32),
    )(z3, weight)
    return idx3.reshape(B, _HW)


def _make_sc_gather(B):
    n_workers = 32
    b_per_w = B // n_workers
    mesh = plsc.VectorSubcoreMesh(core_axis_name="c", subcore_axis_name="s")

    @functools.partial(
        pl.kernel,
        mesh=mesh,
        compiler_params=pltpu.CompilerParams(needs_layout_passes=False),
        out_type=[
            jax.ShapeDtypeStruct((B * _D * _HW,), jnp.float32),
            jax.ShapeDtypeStruct((B * _D * _HW,), jnp.float32),
        ],
        scratch_types=[
            pltpu.VMEM((_D * _K,), jnp.float32),
            pltpu.VMEM((_HW,), jnp.int32),
            pltpu.VMEM((_D * _HW,), jnp.float32),
        ],
    )
    def sc_gather(w_hbm, idx_hbm, zq_hbm, emb_hbm, w_v, idx_v, out_v):
        wid = lax.axis_index("s") * 2 + lax.axis_index("c")
        pltpu.sync_copy(w_hbm, w_v)

        def point_chunk(j, carry):
            base = j * _LANES
            codes = idx_v[pl.ds(base, _LANES)]
            for d in range(_D):
                vals = plsc.load_gather(w_v, [codes + d * _K])
                out_v[pl.ds(d * _HW + base, _LANES)] = vals
            return carry

        for bb in range(b_per_w):
            b = wid * b_per_w + bb
            pltpu.sync_copy(idx_hbm.at[pl.ds(b * _HW, _HW)], idx_v)
            lax.fori_loop(0, _HW // _LANES, point_chunk, 0)
            pltpu.sync_copy(out_v, zq_hbm.at[pl.ds(b * _D * _HW, _D * _HW)])
            pltpu.sync_copy(out_v, emb_hbm.at[pl.ds(b * _D * _HW, _D * _HW)])

    return sc_gather


def kernel(z_g, weight):
    B, D, H, W = z_g.shape
    z3 = z_g.reshape(B, D, H * W)
    idx = _nearest_indices(z3, weight)
    w_flat = weight.reshape(D * _K)
    zq, emb = _make_sc_gather(B)(w_flat, idx.reshape(B * _HW))
    return (zq.reshape(B, D, H, W), emb.reshape(B, D, H, W))


# speedup vs baseline: 1.9164x; 1.9164x over previous
import functools

import jax
import jax.numpy as jnp
from jax.experimental import pallas as pl


def _vq_body(z_ref, w_ref, zq_ref, *, K):
    z = z_ref[0]
    w = w_ref[...]
    wsq_row = jnp.sum(w * w, axis=0, keepdims=True)
    wsq = jnp.transpose(wsq_row, (1, 0))
    scores = jax.lax.dot_general(
        w, z, (((0,), (0,)), ((), ())),
        preferred_element_type=jnp.float32)
    zsq = jnp.sum(z * z, axis=0, keepdims=True)
    dist = (zsq - 2.0 * scores) + wsq
    mind = jnp.min(dist, axis=0, keepdims=True)
    iota = jax.lax.broadcasted_iota(jnp.int32, dist.shape, 0)
    cand = jnp.where(dist == mind, iota, K)
    idx = jnp.min(cand, axis=0, keepdims=True)
    onehot = (iota == idx).astype(jnp.float32)
    q = jax.lax.dot_general(
        w, onehot, (((1,), (0,)), ((), ())),
        precision=jax.lax.Precision.HIGHEST,
        preferred_element_type=jnp.float32)
    zq_ref[0] = q


def kernel(z_g, weight):
    B, D, H, W = z_g.shape
    K = weight.shape[1]
    HW = H * W
    z3 = z_g.reshape(B, D, HW)
    zq3 = pl.pallas_call(
        functools.partial(_vq_body, K=K),
        grid=(B,),
        in_specs=[
            pl.BlockSpec((1, D, HW), lambda i: (i, 0, 0)),
            pl.BlockSpec((D, K), lambda i: (0, 0)),
        ],
        out_specs=pl.BlockSpec((1, D, HW), lambda i: (i, 0, 0)),
        out_shape=jax.ShapeDtypeStruct((B, D, HW), jnp.float32),
    )(z3, weight)
    q = zq3.reshape(B, D, H, W)
    return (q, q)
